# Initial kernel scaffold; baseline (speedup 1.0000x reference)
import jax
import jax.numpy as jnp
from jax.experimental import pallas as pl


def kernel(inputs, embedding):
    raise NotImplementedError("write your pallas kernel here")


# speedup vs baseline: 1.9089x; 1.9089x over previous
import functools

import jax
import jax.numpy as jnp
from jax.experimental import pallas as pl
from jax.experimental.pallas import tpu as pltpu

NUM_EMBEDDINGS = 1024
EMBEDDING_DIM = 30
COMMITMENT_COST = 0.25

BLOCK_ROWS = 512


def _vq_body(n_total, num_blocks, x_ref, emb_ref,
             dist_ref, enc_ref, quant_ref, idx_ref, loss_ref, perp_ref,
             counts_ref, sse_ref):
    i = pl.program_id(0)
    x = x_ref[...]
    emb = emb_ref[...]
    b, d = x.shape
    k = emb.shape[0]

    xsq = jnp.sum(x * x, axis=1, keepdims=True)
    esq = jnp.sum(emb * emb, axis=1)
    dot = jax.lax.dot_general(x, emb, (((1,), (1,)), ((), ())),
                              preferred_element_type=jnp.float32)
    dist = xsq + esq[None, :] - 2.0 * dot
    dist_ref[...] = dist

    min_val = jnp.min(dist, axis=1, keepdims=True)
    iota = jax.lax.broadcasted_iota(jnp.int32, (b, k), 1)
    idx = jnp.min(jnp.where(dist == min_val, iota, k), axis=1)
    idx_ref[...] = idx[:, None].astype(jnp.int32)

    onehot = (iota == idx[:, None]).astype(jnp.float32)
    enc_ref[...] = onehot

    quant = jax.lax.dot_general(onehot, emb, (((1,), (0,)), ((), ())),
                                preferred_element_type=jnp.float32)
    quant_ref[...] = x + (quant - x)
    diff = quant - x

    @pl.when(i == 0)
    def _init():
        sse_ref[...] = jnp.zeros_like(sse_ref)
        counts_ref[...] = jnp.zeros_like(counts_ref)

    sse_ref[...] += jnp.sum(diff * diff, keepdims=True)
    counts_ref[...] += jnp.sum(onehot, axis=0, keepdims=True)

    @pl.when(i == num_blocks - 1)
    def _fin():
        loss_ref[...] = sse_ref[...] / (n_total * d)
        avg = counts_ref[...] / n_total
        perp_ref[...] = jnp.exp(
            -jnp.sum(avg * jnp.log(avg + 1e-10), keepdims=True))


def kernel(inputs, embedding):
    x = jnp.transpose(inputs, (2, 1, 0))
    time, c, batch = x.shape
    flat = x.reshape(-1, EMBEDDING_DIM)
    n = flat.shape[0]
    k = NUM_EMBEDDINGS
    num_blocks = n // BLOCK_ROWS

    body = functools.partial(_vq_body, n, num_blocks)
    dist, enc, quant, idx, loss, perp = pl.pallas_call(
        body,
        grid=(num_blocks,),
        in_specs=[
            pl.BlockSpec((BLOCK_ROWS, EMBEDDING_DIM), lambda i: (i, 0)),
            pl.BlockSpec((k, EMBEDDING_DIM), lambda i: (0, 0)),
        ],
        out_specs=[
            pl.BlockSpec((BLOCK_ROWS, k), lambda i: (i, 0)),
            pl.BlockSpec((BLOCK_ROWS, k), lambda i: (i, 0)),
            pl.BlockSpec((BLOCK_ROWS, EMBEDDING_DIM), lambda i: (i, 0)),
            pl.BlockSpec((BLOCK_ROWS, 1), lambda i: (i, 0)),
            pl.BlockSpec((1, 1), lambda i: (0, 0)),
            pl.BlockSpec((1, 1), lambda i: (0, 0)),
        ],
        out_shape=[
            jax.ShapeDtypeStruct((n, k), jnp.float32),
            jax.ShapeDtypeStruct((n, k), jnp.float32),
            jax.ShapeDtypeStruct((n, EMBEDDING_DIM), jnp.float32),
            jax.ShapeDtypeStruct((n, 1), jnp.int32),
            jax.ShapeDtypeStruct((1, 1), jnp.float32),
            jax.ShapeDtypeStruct((1, 1), jnp.float32),
        ],
        scratch_shapes=[
            pltpu.VMEM((1, k), jnp.float32),
            pltpu.VMEM((1, 1), jnp.float32),
        ],
        compiler_params=pltpu.CompilerParams(
            dimension_semantics=("arbitrary",),
        ),
    )(flat, embedding)

    quantized_st = jnp.transpose(quant.reshape(time, c, batch), (2, 1, 0))
    return (loss[0, 0],
            quantized_st,
            perp[0, 0],
            enc.reshape(batch, c, -1),
            dist.reshape(batch, c, -1),
            idx)


# speedup vs baseline: 1.9850x; 1.0399x over previous
import functools

import jax
import jax.numpy as jnp
from jax.experimental import pallas as pl
from jax.experimental.pallas import tpu as pltpu

NUM_EMBEDDINGS = 1024
EMBEDDING_DIM = 30
COMMITMENT_COST = 0.25

BLOCK_ROWS = 1024


def _vq_body(n_total, num_blocks, x_ref, emb_ref,
             dist_ref, enc_ref, quant_ref, idx_ref, loss_ref, perp_ref,
             counts_ref, sse_ref):
    i = pl.program_id(0)
    x = x_ref[...]
    emb = emb_ref[...]
    b, d = x.shape
    k = emb.shape[0]

    xsq = jnp.sum(x * x, axis=1, keepdims=True)
    esq = jnp.sum(emb * emb, axis=1)
    dot = jax.lax.dot_general(x, emb, (((1,), (1,)), ((), ())),
                              preferred_element_type=jnp.float32)
    dist = xsq + esq[None, :] - 2.0 * dot
    dist_ref[...] = dist

    min_val = jnp.min(dist, axis=1, keepdims=True)
    iota = jax.lax.broadcasted_iota(jnp.int32, (b, k), 1)
    idx = jnp.min(jnp.where(dist == min_val, iota, k), axis=1)
    idx_ref[...] = idx[:, None].astype(jnp.int32)

    onehot = (iota == idx[:, None]).astype(jnp.float32)
    enc_ref[...] = onehot

    quant = jax.lax.dot_general(onehot, emb, (((1,), (0,)), ((), ())),
                                preferred_element_type=jnp.float32)
    quant_ref[...] = x + (quant - x)
    diff = quant - x

    @pl.when(i == 0)
    def _init():
        sse_ref[...] = jnp.zeros_like(sse_ref)
        counts_ref[...] = jnp.zeros_like(counts_ref)

    sse_ref[...] += jnp.sum(diff * diff, keepdims=True)
    counts_ref[...] += jnp.sum(onehot, axis=0, keepdims=True)

    @pl.when(i == num_blocks - 1)
    def _fin():
        loss_ref[...] = sse_ref[...] / (n_total * d)
        avg = counts_ref[...] / n_total
        perp_ref[...] = jnp.exp(
            -jnp.sum(avg * jnp.log(avg + 1e-10), keepdims=True))


def kernel(inputs, embedding):
    x = jnp.transpose(inputs, (2, 1, 0))
    time, c, batch = x.shape
    flat = x.reshape(-1, EMBEDDING_DIM)
    n = flat.shape[0]
    k = NUM_EMBEDDINGS
    num_blocks = n // BLOCK_ROWS

    body = functools.partial(_vq_body, n, num_blocks)
    dist, enc, quant, idx, loss, perp = pl.pallas_call(
        body,
        grid=(num_blocks,),
        in_specs=[
            pl.BlockSpec((BLOCK_ROWS, EMBEDDING_DIM), lambda i: (i, 0)),
            pl.BlockSpec((k, EMBEDDING_DIM), lambda i: (0, 0)),
        ],
        out_specs=[
            pl.BlockSpec((BLOCK_ROWS, k), lambda i: (i, 0)),
            pl.BlockSpec((BLOCK_ROWS, k), lambda i: (i, 0)),
            pl.BlockSpec((BLOCK_ROWS, EMBEDDING_DIM), lambda i: (i, 0)),
            pl.BlockSpec((BLOCK_ROWS, 1), lambda i: (i, 0)),
            pl.BlockSpec((1, 1), lambda i: (0, 0)),
            pl.BlockSpec((1, 1), lambda i: (0, 0)),
        ],
        out_shape=[
            jax.ShapeDtypeStruct((n, k), jnp.float32),
            jax.ShapeDtypeStruct((n, k), jnp.float32),
            jax.ShapeDtypeStruct((n, EMBEDDING_DIM), jnp.float32),
            jax.ShapeDtypeStruct((n, 1), jnp.int32),
            jax.ShapeDtypeStruct((1, 1), jnp.float32),
            jax.ShapeDtypeStruct((1, 1), jnp.float32),
        ],
        scratch_shapes=[
            pltpu.VMEM((1, k), jnp.float32),
            pltpu.VMEM((1, 1), jnp.float32),
        ],
        compiler_params=pltpu.CompilerParams(
            dimension_semantics=("arbitrary",),
        ),
    )(flat, embedding)

    quantized_st = jnp.transpose(quant.reshape(time, c, batch), (2, 1, 0))
    return (loss[0, 0],
            quantized_st,
            perp[0, 0],
            enc.reshape(batch, c, -1),
            dist.reshape(batch, c, -1),
            idx)


# speedup vs baseline: 2.0131x; 1.0141x over previous
import functools

import jax
import jax.numpy as jnp
from jax.experimental import pallas as pl
from jax.experimental.pallas import tpu as pltpu

NUM_EMBEDDINGS = 1024
EMBEDDING_DIM = 30
COMMITMENT_COST = 0.25

BLOCK_ROWS = 2304


def _vq_body(n_total, num_blocks, x_ref, emb_ref,
             dist_ref, enc_ref, quant_ref, idx_ref, loss_ref, perp_ref,
             counts_ref, sse_ref):
    i = pl.program_id(0)
    x = x_ref[...]
    emb = emb_ref[...]
    b, d = x.shape
    k = emb.shape[0]

    xsq = jnp.sum(x * x, axis=1, keepdims=True)
    esq = jnp.sum(emb * emb, axis=1)
    dot = jax.lax.dot_general(x, emb, (((1,), (1,)), ((), ())),
                              preferred_element_type=jnp.float32)
    dist = xsq + esq[None, :] - 2.0 * dot
    dist_ref[...] = dist

    min_val = jnp.min(dist, axis=1, keepdims=True)
    iota = jax.lax.broadcasted_iota(jnp.int32, (b, k), 1)
    idx = jnp.min(jnp.where(dist == min_val, iota, k), axis=1)
    idx_ref[...] = idx[:, None].astype(jnp.int32)

    onehot = (iota == idx[:, None]).astype(jnp.float32)
    enc_ref[...] = onehot

    quant = jax.lax.dot_general(onehot, emb, (((1,), (0,)), ((), ())),
                                preferred_element_type=jnp.float32)
    quant_ref[...] = x + (quant - x)
    diff = quant - x

    @pl.when(i == 0)
    def _init():
        sse_ref[...] = jnp.zeros_like(sse_ref)
        counts_ref[...] = jnp.zeros_like(counts_ref)

    sse_ref[...] += jnp.sum(diff * diff, keepdims=True)
    counts_ref[...] += jnp.sum(onehot, axis=0, keepdims=True)

    @pl.when(i == num_blocks - 1)
    def _fin():
        loss_ref[...] = sse_ref[...] / (n_total * d)
        avg = counts_ref[...] / n_total
        perp_ref[...] = jnp.exp(
            -jnp.sum(avg * jnp.log(avg + 1e-10), keepdims=True))


def kernel(inputs, embedding):
    x = jnp.transpose(inputs, (2, 1, 0))
    time, c, batch = x.shape
    flat = x.reshape(-1, EMBEDDING_DIM)
    n = flat.shape[0]
    k = NUM_EMBEDDINGS
    num_blocks = n // BLOCK_ROWS

    body = functools.partial(_vq_body, n, num_blocks)
    dist, enc, quant, idx, loss, perp = pl.pallas_call(
        body,
        grid=(num_blocks,),
        in_specs=[
            pl.BlockSpec((BLOCK_ROWS, EMBEDDING_DIM), lambda i: (i, 0)),
            pl.BlockSpec((k, EMBEDDING_DIM), lambda i: (0, 0)),
        ],
        out_specs=[
            pl.BlockSpec((BLOCK_ROWS, k), lambda i: (i, 0)),
            pl.BlockSpec((BLOCK_ROWS, k), lambda i: (i, 0)),
            pl.BlockSpec((BLOCK_ROWS, EMBEDDING_DIM), lambda i: (i, 0)),
            pl.BlockSpec((BLOCK_ROWS, 1), lambda i: (i, 0)),
            pl.BlockSpec((1, 1), lambda i: (0, 0)),
            pl.BlockSpec((1, 1), lambda i: (0, 0)),
        ],
        out_shape=[
            jax.ShapeDtypeStruct((n, k), jnp.float32),
            jax.ShapeDtypeStruct((n, k), jnp.float32),
            jax.ShapeDtypeStruct((n, EMBEDDING_DIM), jnp.float32),
            jax.ShapeDtypeStruct((n, 1), jnp.int32),
            jax.ShapeDtypeStruct((1, 1), jnp.float32),
            jax.ShapeDtypeStruct((1, 1), jnp.float32),
        ],
        scratch_shapes=[
            pltpu.VMEM((1, k), jnp.float32),
            pltpu.VMEM((1, 1), jnp.float32),
        ],
        compiler_params=pltpu.CompilerParams(
            dimension_semantics=("arbitrary",),
        ),
    )(flat, embedding)

    quantized_st = jnp.transpose(quant.reshape(time, c, batch), (2, 1, 0))
    return (loss[0, 0],
            quantized_st,
            perp[0, 0],
            enc.reshape(batch, c, -1),
            dist.reshape(batch, c, -1),
            idx)


# speedup vs baseline: 2.6481x; 1.3154x over previous
import functools

import numpy as np

import jax
import jax.numpy as jnp
from jax.experimental import pallas as pl
from jax.experimental.pallas import tpu as pltpu
from jax.sharding import Mesh, PartitionSpec as P
from jax import shard_map

NUM_EMBEDDINGS = 1024
EMBEDDING_DIM = 30
COMMITMENT_COST = 0.25

BLOCK_ROWS = 2304


def _vq_body(num_blocks, x_ref, emb_ref,
             dist_ref, enc_ref, quant_ref, idx_ref, sse_ref, counts_ref):
    i = pl.program_id(0)
    x = x_ref[...]
    emb = emb_ref[...]
    b, d = x.shape
    k = emb.shape[0]

    xsq = jnp.sum(x * x, axis=1, keepdims=True)
    esq = jnp.sum(emb * emb, axis=1)
    dot = jax.lax.dot_general(x, emb, (((1,), (1,)), ((), ())),
                              preferred_element_type=jnp.float32)
    dist = xsq + esq[None, :] - 2.0 * dot
    dist_ref[...] = dist

    min_val = jnp.min(dist, axis=1, keepdims=True)
    iota = jax.lax.broadcasted_iota(jnp.int32, (b, k), 1)
    idx = jnp.min(jnp.where(dist == min_val, iota, k), axis=1)
    idx_ref[...] = idx[:, None].astype(jnp.int32)

    onehot = (iota == idx[:, None]).astype(jnp.float32)
    enc_ref[...] = onehot

    quant = jax.lax.dot_general(onehot, emb, (((1,), (0,)), ((), ())),
                                preferred_element_type=jnp.float32)
    quant_ref[...] = x + (quant - x)
    diff = quant - x

    @pl.when(i == 0)
    def _init():
        sse_ref[...] = jnp.zeros_like(sse_ref)
        counts_ref[...] = jnp.zeros_like(counts_ref)

    sse_ref[...] += jnp.sum(diff * diff, keepdims=True)
    counts_ref[...] += jnp.sum(onehot, axis=0, keepdims=True)


def _vq_shard(n_total, x_loc, embedding):
    flat = x_loc.reshape(-1, EMBEDDING_DIM)
    n = flat.shape[0]
    k = NUM_EMBEDDINGS
    block = min(BLOCK_ROWS, n)
    num_blocks = n // block

    body = functools.partial(_vq_body, num_blocks)
    dist, enc, quant, idx, sse, counts = pl.pallas_call(
        body,
        grid=(num_blocks,),
        in_specs=[
            pl.BlockSpec((block, EMBEDDING_DIM), lambda i: (i, 0)),
            pl.BlockSpec((k, EMBEDDING_DIM), lambda i: (0, 0)),
        ],
        out_specs=[
            pl.BlockSpec((block, k), lambda i: (i, 0)),
            pl.BlockSpec((block, k), lambda i: (i, 0)),
            pl.BlockSpec((block, EMBEDDING_DIM), lambda i: (i, 0)),
            pl.BlockSpec((block, 1), lambda i: (i, 0)),
            pl.BlockSpec((1, 1), lambda i: (0, 0)),
            pl.BlockSpec((1, k), lambda i: (0, 0)),
        ],
        out_shape=[
            jax.ShapeDtypeStruct((n, k), jnp.float32),
            jax.ShapeDtypeStruct((n, k), jnp.float32),
            jax.ShapeDtypeStruct((n, EMBEDDING_DIM), jnp.float32),
            jax.ShapeDtypeStruct((n, 1), jnp.int32),
            jax.ShapeDtypeStruct((1, 1), jnp.float32),
            jax.ShapeDtypeStruct((1, k), jnp.float32),
        ],
        compiler_params=pltpu.CompilerParams(
            dimension_semantics=("arbitrary",),
        ),
    )(flat, embedding)

    sse = jax.lax.psum(sse, "x")
    counts = jax.lax.psum(counts, "x")
    loss = sse[0, 0] / (n_total * EMBEDDING_DIM)
    avg = counts[0] / n_total
    perp = jnp.exp(-jnp.sum(avg * jnp.log(avg + 1e-10)))
    return dist, enc, quant, idx, loss, perp


def kernel(inputs, embedding):
    x = jnp.transpose(inputs, (2, 1, 0))
    time, c, batch = x.shape
    n_total = time * c

    devs = jax.devices()
    n_dev = 2 if len(devs) >= 2 and time % 2 == 0 else 1
    mesh = Mesh(np.array(devs[:n_dev]), ("x",))

    shard_fn = shard_map(
        functools.partial(_vq_shard, n_total),
        mesh=mesh,
        in_specs=(P("x", None, None), P(None, None)),
        out_specs=(P("x", None), P("x", None), P("x", None), P("x", None),
                   P(), P()),
        check_vma=False,
    )
    dist, enc, quant, idx, loss, perp = shard_fn(x, embedding)

    quantized_st = jnp.transpose(quant.reshape(time, c, batch), (2, 1, 0))
    return (loss,
            quantized_st,
            perp,
            enc.reshape(batch, c, -1),
            dist.reshape(batch, c, -1),
            idx)
